# baseline (speedup 1.0000x reference)
import math

import jax
import jax.numpy as jnp
from jax.experimental import pallas as pl
from jax.experimental.pallas import tpu as pltpu

_EPS_BN = 1e-5
_K = 10
_N = 100
_NP = 104
_G = 4
_NEG = float("-inf")


def _lrelu(v):
    return jnp.where(v >= 0, v, 0.2 * v)


def _dotd(a, b):
    return jnp.dot(
        a.astype(jnp.bfloat16),
        b.astype(jnp.bfloat16),
        preferred_element_type=jnp.float32,
    )


def _limbs(y):
    hi = y.astype(jnp.bfloat16)
    r = y - hi.astype(jnp.float32)
    lo2 = r.astype(jnp.bfloat16)
    lo3 = (r - lo2.astype(jnp.float32)).astype(jnp.bfloat16)
    return jnp.concatenate([hi, lo2, lo3], axis=1)


def _stage(y, jjf, a_w, b_w, gs, bias, w2, gs2, b2):
    yb = y.astype(jnp.bfloat16)
    inner = jax.lax.dot_general(
        yb, yb, (((1,), (1,)), ((), ())), preferred_element_type=jnp.float32
    )
    norms = jnp.transpose(jnp.sum(y * y, axis=1, keepdims=True))
    pd = 2.0 * inner - norms
    lbs = _limbs(y)
    zrow = jnp.zeros((_NP - _N, 32), jnp.float32)
    parts = []
    for _ in range(_K):
        m = jnp.max(pd, axis=1, keepdims=True)
        hit = pd >= m
        idx = jnp.min(jnp.where(hit, jjf, float(_N)), axis=1, keepdims=True)
        sel = jjf == idx
        pd = jnp.where(sel, _NEG, pd)
        oh = jnp.where(sel, 1.0, 0.0).astype(jnp.bfloat16)
        lim = jnp.dot(oh, lbs, preferred_element_type=jnp.float32)
        fk = (lim[:, :32] + lim[:, 32:64]) + lim[:, 64:]
        parts.append(fk - y)
        parts.append(zrow)
    d = jnp.concatenate(parts, axis=0).reshape(_K, _NP, 32)
    base = jnp.concatenate([_dotd(y, b_w), zrow], axis=0)
    t = _dotd(d.reshape(_K * _NP, 32), a_w).reshape(_K, _NP, 32)
    h = _lrelu((t + base[None]) * gs + bias)
    if w2 is not None:
        t2 = _dotd(h.reshape(_K * _NP, 32), w2).reshape(_K, _NP, 32)
        h = _lrelu(t2 * gs2 + b2)
    return jnp.max(h, axis=0)[:_N]


def _dgcnn_kernel(
    x_ref,
    w0_ref, g0_ref, b0_ref,
    a1_ref, c1_ref, g1_ref, b1_ref, w2_ref, g2_ref, b2_ref,
    a3_ref, c3_ref, g3_ref, b3_ref, w4_ref, g4_ref, b4_ref,
    a5_ref, c5_ref, g5_ref, b5_ref,
    w6_ref, g6_ref, b6_ref,
    w7g_ref, w7abc_ref, g7_ref, b7_ref,
    w8_ref, g8_ref, b8_ref, w9_ref,
    out_ref,
):
    jjf = jax.lax.broadcasted_iota(jnp.int32, (_N, _N), 1).astype(jnp.float32)
    for g in range(_G):
        x = x_ref[g]
        y0 = _lrelu(_dotd(x, w0_ref[:]) * g0_ref[:] + b0_ref[:])
        x1 = _stage(y0, jjf, a1_ref[:], c1_ref[:], g1_ref[:], b1_ref[:],
                    w2_ref[:], g2_ref[:], b2_ref[:])
        x2 = _stage(x1, jjf, a3_ref[:], c3_ref[:], g3_ref[:], b3_ref[:],
                    w4_ref[:], g4_ref[:], b4_ref[:])
        x3 = _stage(x2, jjf, a5_ref[:], c5_ref[:], g5_ref[:], b5_ref[:],
                    None, None, None)
        xc = jnp.concatenate([x1, x2, x3], axis=1)
        h6 = _lrelu(_dotd(xc, w6_ref[:]) * g6_ref[:] + b6_ref[:])
        g6 = jnp.max(h6, axis=0, keepdims=True)
        h7 = _lrelu(
            (_dotd(g6, w7g_ref[:]) + _dotd(xc, w7abc_ref[:]))
            * g7_ref[:] + b7_ref[:]
        )
        h8 = _lrelu(_dotd(h7, w8_ref[:]) * g8_ref[:] + b8_ref[:])
        out_ref[g] = _dotd(h8, w9_ref[:])


def kernel(obs, params):
    p = params
    s = 1.0 / math.sqrt(1.0 + _EPS_BN)

    def gs(name):
        return (p["g" + name] * s)[None, :]

    def bias(name):
        return p["b" + name][None, :]

    w0 = p["W0"].T
    w1 = p["W1"].T
    w2 = p["W2"].T
    w3 = p["W3"].T
    w4 = p["W4"].T
    w5 = p["W5"].T
    w6 = p["W6"].T
    w7 = p["W7"].T
    w8 = p["W8"].T
    w9 = p["W9"].T

    weights = [
        w0, gs("0"), bias("0"),
        w1[:32], w1[32:], gs("1"), bias("1"), w2, gs("2"), bias("2"),
        w3[:32], w3[32:], gs("3"), bias("3"), w4, gs("4"), bias("4"),
        w5[:32], w5[32:], gs("5"), bias("5"),
        w6, gs("6"), bias("6"),
        w7[:512], w7[512:], gs("7"), bias("7"),
        w8, gs("8"), bias("8"), w9,
    ]

    b = obs.shape[0]
    obs3 = obs.reshape(b, _N, obs.shape[1] // _N)

    in_specs = [pl.BlockSpec((_G, _N, obs3.shape[2]), lambda i: (i, 0, 0))]
    for w in weights:
        in_specs.append(
            pl.BlockSpec(w.shape, (lambda nd: (lambda i: (0,) * nd))(w.ndim))
        )

    out = pl.pallas_call(
        _dgcnn_kernel,
        grid=(b // _G,),
        in_specs=in_specs,
        out_specs=pl.BlockSpec((_G, _N, 1), lambda i: (i, 0, 0)),
        out_shape=jax.ShapeDtypeStruct((b, _N, 1), jnp.float32),
        compiler_params=pltpu.CompilerParams(
            dimension_semantics=("parallel",)
        ),
    )(obs3, *weights)

    q = out.reshape(b, _N)
    return q[None, :, None, :]


# speedup vs baseline: 1.0686x; 1.0686x over previous
import math

import jax
import jax.numpy as jnp
from jax.experimental import pallas as pl
from jax.experimental.pallas import tpu as pltpu

_EPS_BN = 1e-5
_K = 10
_N = 100
_NP = 104
_G = 4
_NEG = float("-inf")


def _lrelu(v):
    return jnp.maximum(v, 0.2 * v)


def _dotd(a, b):
    return jnp.dot(
        a.astype(jnp.bfloat16),
        b.astype(jnp.bfloat16),
        preferred_element_type=jnp.float32,
    )


def _limbs(y):
    hi = y.astype(jnp.bfloat16)
    r = y - hi.astype(jnp.float32)
    lo2 = r.astype(jnp.bfloat16)
    lo3 = (r - lo2.astype(jnp.float32)).astype(jnp.bfloat16)
    return jnp.concatenate([hi, lo2, lo3], axis=1)


def _stage(y, jjf, a_w, b_w, gs, bias, w2, gs2, b2):
    yb = y.astype(jnp.bfloat16)
    inner = jax.lax.dot_general(
        yb, yb, (((1,), (1,)), ((), ())), preferred_element_type=jnp.float32
    )
    norms = jnp.transpose(jnp.sum(y * y, axis=1, keepdims=True))
    pd = 2.0 * inner - norms
    sels = []
    for _ in range(_K):
        m = jnp.max(pd, axis=1, keepdims=True)
        hit = pd >= m
        idx = jnp.min(jnp.where(hit, jjf, float(_N)), axis=1, keepdims=True)
        sel = jjf == idx
        pd = jnp.where(sel, _NEG, pd)
        sels.append(jnp.where(sel, 1.0, 0.0))
    zpad = jnp.zeros((_NP - _N, _N), jnp.float32)
    msel = jnp.concatenate(
        [a for s in sels for a in (s, zpad)], axis=0
    )
    lim = jnp.dot(
        msel.astype(jnp.bfloat16), _limbs(y),
        preferred_element_type=jnp.float32,
    )
    feat = (lim[:, :32] + lim[:, 32:64]) + lim[:, 64:]
    zrow = jnp.zeros((_NP - _N, 32), jnp.float32)
    ypad = jnp.concatenate([y, zrow], axis=0)
    d = feat.reshape(_K, _NP, 32) - ypad[None]
    base = jnp.concatenate([_dotd(y, b_w), zrow], axis=0)
    t = _dotd(d.reshape(_K * _NP, 32), a_w).reshape(_K, _NP, 32)
    h = _lrelu((t + base[None]) * gs + bias)
    if w2 is not None:
        t2 = _dotd(h.reshape(_K * _NP, 32), w2).reshape(_K, _NP, 32)
        h = _lrelu(t2 * gs2 + b2)
    return jnp.max(h, axis=0)[:_N]


def _dgcnn_kernel(
    x_ref,
    w0_ref, g0_ref, b0_ref,
    a1_ref, c1_ref, g1_ref, b1_ref, w2_ref, g2_ref, b2_ref,
    a3_ref, c3_ref, g3_ref, b3_ref, w4_ref, g4_ref, b4_ref,
    a5_ref, c5_ref, g5_ref, b5_ref,
    w6_ref, g6_ref, b6_ref,
    w7g_ref, w7abc_ref, g7_ref, b7_ref,
    w8_ref, g8_ref, b8_ref, w9_ref,
    out_ref,
):
    jjf = jax.lax.broadcasted_iota(jnp.int32, (_N, _N), 1).astype(jnp.float32)
    for g in range(_G):
        x = x_ref[g]
        y0 = _lrelu(_dotd(x, w0_ref[:]) * g0_ref[:] + b0_ref[:])
        x1 = _stage(y0, jjf, a1_ref[:], c1_ref[:], g1_ref[:], b1_ref[:],
                    w2_ref[:], g2_ref[:], b2_ref[:])
        x2 = _stage(x1, jjf, a3_ref[:], c3_ref[:], g3_ref[:], b3_ref[:],
                    w4_ref[:], g4_ref[:], b4_ref[:])
        x3 = _stage(x2, jjf, a5_ref[:], c5_ref[:], g5_ref[:], b5_ref[:],
                    None, None, None)
        xc = jnp.concatenate([x1, x2, x3], axis=1)
        h6 = _lrelu(_dotd(xc, w6_ref[:]) * g6_ref[:] + b6_ref[:])
        g6 = jnp.max(h6, axis=0, keepdims=True)
        h7 = _lrelu(
            (_dotd(g6, w7g_ref[:]) + _dotd(xc, w7abc_ref[:]))
            * g7_ref[:] + b7_ref[:]
        )
        h8 = _lrelu(_dotd(h7, w8_ref[:]) * g8_ref[:] + b8_ref[:])
        out_ref[g] = _dotd(h8, w9_ref[:])


def kernel(obs, params):
    p = params
    s = 1.0 / math.sqrt(1.0 + _EPS_BN)

    def gs(name):
        return (p["g" + name] * s)[None, :]

    def bias(name):
        return p["b" + name][None, :]

    w0 = p["W0"].T
    w1 = p["W1"].T
    w2 = p["W2"].T
    w3 = p["W3"].T
    w4 = p["W4"].T
    w5 = p["W5"].T
    w6 = p["W6"].T
    w7 = p["W7"].T
    w8 = p["W8"].T
    w9 = p["W9"].T

    weights = [
        w0, gs("0"), bias("0"),
        w1[:32], w1[32:], gs("1"), bias("1"), w2, gs("2"), bias("2"),
        w3[:32], w3[32:], gs("3"), bias("3"), w4, gs("4"), bias("4"),
        w5[:32], w5[32:], gs("5"), bias("5"),
        w6, gs("6"), bias("6"),
        w7[:512], w7[512:], gs("7"), bias("7"),
        w8, gs("8"), bias("8"), w9,
    ]

    b = obs.shape[0]
    obs3 = obs.reshape(b, _N, obs.shape[1] // _N)

    in_specs = [pl.BlockSpec((_G, _N, obs3.shape[2]), lambda i: (i, 0, 0))]
    for w in weights:
        in_specs.append(
            pl.BlockSpec(w.shape, (lambda nd: (lambda i: (0,) * nd))(w.ndim))
        )

    out = pl.pallas_call(
        _dgcnn_kernel,
        grid=(b // _G,),
        in_specs=in_specs,
        out_specs=pl.BlockSpec((_G, _N, 1), lambda i: (i, 0, 0)),
        out_shape=jax.ShapeDtypeStruct((b, _N, 1), jnp.float32),
        compiler_params=pltpu.CompilerParams(
            dimension_semantics=("parallel",)
        ),
    )(obs3, *weights)

    q = out.reshape(b, _N)
    return q[None, :, None, :]


# speedup vs baseline: 1.1327x; 1.0600x over previous
import math

import jax
import jax.numpy as jnp
from jax.experimental import pallas as pl
from jax.experimental.pallas import tpu as pltpu

_EPS_BN = 1e-5
_K = 10
_N = 100
_NP = 104
_G = 4
_NEG = float("-inf")


def _lrelu(v):
    return jnp.maximum(v, 0.2 * v)


def _dotd(a, b):
    return jnp.dot(
        a.astype(jnp.bfloat16),
        b.astype(jnp.bfloat16),
        preferred_element_type=jnp.float32,
    )


def _limbs(y):
    hi = y.astype(jnp.bfloat16)
    r = y - hi.astype(jnp.float32)
    lo2 = r.astype(jnp.bfloat16)
    lo3 = (r - lo2.astype(jnp.float32)).astype(jnp.bfloat16)
    return jnp.concatenate([hi, lo2, lo3], axis=1)


def _knn_onehots(ys, jjf):
    zpad = jnp.zeros((_NP - _N, _N), jnp.float32)
    pds = []
    for y in ys:
        yb = y.astype(jnp.bfloat16)
        inner = jax.lax.dot_general(
            yb, yb, (((1,), (1,)), ((), ())),
            preferred_element_type=jnp.float32,
        )
        norms = jnp.transpose(jnp.sum(y * y, axis=1, keepdims=True))
        pds.append(jnp.concatenate([2.0 * inner - norms, zpad], axis=0))
    pdb = jnp.concatenate(pds, axis=0)
    sels = []
    for _ in range(_K):
        m = jnp.max(pdb, axis=1, keepdims=True)
        hit = pdb >= m
        idx = jnp.min(jnp.where(hit, jjf, float(_N)), axis=1, keepdims=True)
        sel = jjf == idx
        pdb = jnp.where(sel, _NEG, pdb)
        sels.append(jnp.where(sel, 1.0, 0.0))
    return [
        jnp.concatenate([s[g * _NP:(g + 1) * _NP] for s in sels], axis=0)
        for g in range(_G)
    ]


def _edge_conv(y, msel, a_w, b_w, gs, bias, w2, gs2, b2):
    lim = jnp.dot(
        msel.astype(jnp.bfloat16), _limbs(y),
        preferred_element_type=jnp.float32,
    )
    feat = (lim[:, :32] + lim[:, 32:64]) + lim[:, 64:]
    zrow = jnp.zeros((_NP - _N, 32), jnp.float32)
    ypad = jnp.concatenate([y, zrow], axis=0)
    d = feat.reshape(_K, _NP, 32) - ypad[None]
    base = jnp.concatenate([_dotd(y, b_w), zrow], axis=0)
    t = _dotd(d.reshape(_K * _NP, 32), a_w).reshape(_K, _NP, 32)
    h = _lrelu((t + base[None]) * gs + bias)
    if w2 is not None:
        t2 = _dotd(h.reshape(_K * _NP, 32), w2).reshape(_K, _NP, 32)
        h = _lrelu(t2 * gs2 + b2)
    return jnp.max(h, axis=0)[:_N]


def _dgcnn_kernel(
    x_ref,
    w0_ref, g0_ref, b0_ref,
    a1_ref, c1_ref, g1_ref, b1_ref, w2_ref, g2_ref, b2_ref,
    a3_ref, c3_ref, g3_ref, b3_ref, w4_ref, g4_ref, b4_ref,
    a5_ref, c5_ref, g5_ref, b5_ref,
    w6_ref, g6_ref, b6_ref,
    w7g_ref, w7abc_ref, g7_ref, b7_ref,
    w8_ref, g8_ref, b8_ref, w9_ref,
    out_ref,
):
    jjf = jax.lax.broadcasted_iota(
        jnp.int32, (_G * _NP, _N), 1
    ).astype(jnp.float32)
    y0 = [
        _lrelu(_dotd(x_ref[g], w0_ref[:]) * g0_ref[:] + b0_ref[:])
        for g in range(_G)
    ]
    m1 = _knn_onehots(y0, jjf)
    x1 = [
        _edge_conv(y0[g], m1[g], a1_ref[:], c1_ref[:], g1_ref[:], b1_ref[:],
                   w2_ref[:], g2_ref[:], b2_ref[:])
        for g in range(_G)
    ]
    m2 = _knn_onehots(x1, jjf)
    x2 = [
        _edge_conv(x1[g], m2[g], a3_ref[:], c3_ref[:], g3_ref[:], b3_ref[:],
                   w4_ref[:], g4_ref[:], b4_ref[:])
        for g in range(_G)
    ]
    m3 = _knn_onehots(x2, jjf)
    x3 = [
        _edge_conv(x2[g], m3[g], a5_ref[:], c5_ref[:], g5_ref[:], b5_ref[:],
                   None, None, None)
        for g in range(_G)
    ]
    for g in range(_G):
        xc = jnp.concatenate([x1[g], x2[g], x3[g]], axis=1)
        h6 = _lrelu(_dotd(xc, w6_ref[:]) * g6_ref[:] + b6_ref[:])
        g6 = jnp.max(h6, axis=0, keepdims=True)
        h7 = _lrelu(
            (_dotd(g6, w7g_ref[:]) + _dotd(xc, w7abc_ref[:]))
            * g7_ref[:] + b7_ref[:]
        )
        h8 = _lrelu(_dotd(h7, w8_ref[:]) * g8_ref[:] + b8_ref[:])
        out_ref[g] = _dotd(h8, w9_ref[:])


def kernel(obs, params):
    p = params
    s = 1.0 / math.sqrt(1.0 + _EPS_BN)

    def gs(name):
        return (p["g" + name] * s)[None, :]

    def bias(name):
        return p["b" + name][None, :]

    w0 = p["W0"].T
    w1 = p["W1"].T
    w2 = p["W2"].T
    w3 = p["W3"].T
    w4 = p["W4"].T
    w5 = p["W5"].T
    w6 = p["W6"].T
    w7 = p["W7"].T
    w8 = p["W8"].T
    w9 = p["W9"].T

    weights = [
        w0, gs("0"), bias("0"),
        w1[:32], w1[32:], gs("1"), bias("1"), w2, gs("2"), bias("2"),
        w3[:32], w3[32:], gs("3"), bias("3"), w4, gs("4"), bias("4"),
        w5[:32], w5[32:], gs("5"), bias("5"),
        w6, gs("6"), bias("6"),
        w7[:512], w7[512:], gs("7"), bias("7"),
        w8, gs("8"), bias("8"), w9,
    ]

    b = obs.shape[0]
    obs3 = obs.reshape(b, _N, obs.shape[1] // _N)

    in_specs = [pl.BlockSpec((_G, _N, obs3.shape[2]), lambda i: (i, 0, 0))]
    for w in weights:
        in_specs.append(
            pl.BlockSpec(w.shape, (lambda nd: (lambda i: (0,) * nd))(w.ndim))
        )

    out = pl.pallas_call(
        _dgcnn_kernel,
        grid=(b // _G,),
        in_specs=in_specs,
        out_specs=pl.BlockSpec((_G, _N, 1), lambda i: (i, 0, 0)),
        out_shape=jax.ShapeDtypeStruct((b, _N, 1), jnp.float32),
        compiler_params=pltpu.CompilerParams(
            dimension_semantics=("parallel",)
        ),
    )(obs3, *weights)

    q = out.reshape(b, _N)
    return q[None, :, None, :]


# speedup vs baseline: 1.1626x; 1.0264x over previous
import math

import jax
import jax.numpy as jnp
from jax.experimental import pallas as pl
from jax.experimental.pallas import tpu as pltpu

_EPS_BN = 1e-5
_K = 10
_N = 100
_NP = 104
_G = 8
_NEG = float("-inf")


def _lrelu(v):
    return jnp.maximum(v, 0.2 * v)


def _dotd(a, b):
    return jnp.dot(
        a.astype(jnp.bfloat16),
        b.astype(jnp.bfloat16),
        preferred_element_type=jnp.float32,
    )


def _limbs(y):
    hi = y.astype(jnp.bfloat16)
    r = y - hi.astype(jnp.float32)
    lo2 = r.astype(jnp.bfloat16)
    lo3 = (r - lo2.astype(jnp.float32)).astype(jnp.bfloat16)
    return jnp.concatenate([hi, lo2, lo3], axis=1)


def _knn_onehots(ys, jjf):
    zpad = jnp.zeros((_NP - _N, _N), jnp.float32)
    pds = []
    for y in ys:
        yb = y.astype(jnp.bfloat16)
        inner = jax.lax.dot_general(
            yb, yb, (((1,), (1,)), ((), ())),
            preferred_element_type=jnp.float32,
        )
        norms = jnp.transpose(jnp.sum(y * y, axis=1, keepdims=True))
        pds.append(jnp.concatenate([2.0 * inner - norms, zpad], axis=0))
    pdb = jnp.concatenate(pds, axis=0)
    sels = []
    for _ in range(_K):
        m = jnp.max(pdb, axis=1, keepdims=True)
        hit = pdb >= m
        idx = jnp.min(jnp.where(hit, jjf, float(_N)), axis=1, keepdims=True)
        sel = jjf == idx
        pdb = jnp.where(sel, _NEG, pdb)
        sels.append(jnp.where(sel, 1.0, 0.0))
    return [
        jnp.concatenate([s[g * _NP:(g + 1) * _NP] for s in sels], axis=0)
        for g in range(_G)
    ]


def _edge_conv(y, msel, a_w, b_w, gs, bias, w2, gs2, b2):
    lim = jnp.dot(
        msel.astype(jnp.bfloat16), _limbs(y),
        preferred_element_type=jnp.float32,
    )
    feat = (lim[:, :32] + lim[:, 32:64]) + lim[:, 64:]
    zrow = jnp.zeros((_NP - _N, 32), jnp.float32)
    ypad = jnp.concatenate([y, zrow], axis=0)
    d = feat.reshape(_K, _NP, 32) - ypad[None]
    base = jnp.concatenate([_dotd(y, b_w), zrow], axis=0)
    t = _dotd(d.reshape(_K * _NP, 32), a_w).reshape(_K, _NP, 32)
    h = _lrelu((t + base[None]) * gs + bias)
    if w2 is not None:
        t2 = _dotd(h.reshape(_K * _NP, 32), w2).reshape(_K, _NP, 32)
        h = _lrelu(t2 * gs2 + b2)
    return jnp.max(h, axis=0)[:_N]


def _dgcnn_kernel(
    x_ref,
    w0_ref, g0_ref, b0_ref,
    a1_ref, c1_ref, g1_ref, b1_ref, w2_ref, g2_ref, b2_ref,
    a3_ref, c3_ref, g3_ref, b3_ref, w4_ref, g4_ref, b4_ref,
    a5_ref, c5_ref, g5_ref, b5_ref,
    w6_ref, g6_ref, b6_ref,
    w7g_ref, w7abc_ref, g7_ref, b7_ref,
    w8_ref, g8_ref, b8_ref, w9_ref,
    out_ref,
):
    jjf = jax.lax.broadcasted_iota(
        jnp.int32, (_G * _NP, _N), 1
    ).astype(jnp.float32)
    y0 = [
        _lrelu(_dotd(x_ref[g], w0_ref[:]) * g0_ref[:] + b0_ref[:])
        for g in range(_G)
    ]
    m1 = _knn_onehots(y0, jjf)
    x1 = [
        _edge_conv(y0[g], m1[g], a1_ref[:], c1_ref[:], g1_ref[:], b1_ref[:],
                   w2_ref[:], g2_ref[:], b2_ref[:])
        for g in range(_G)
    ]
    m2 = _knn_onehots(x1, jjf)
    x2 = [
        _edge_conv(x1[g], m2[g], a3_ref[:], c3_ref[:], g3_ref[:], b3_ref[:],
                   w4_ref[:], g4_ref[:], b4_ref[:])
        for g in range(_G)
    ]
    m3 = _knn_onehots(x2, jjf)
    x3 = [
        _edge_conv(x2[g], m3[g], a5_ref[:], c5_ref[:], g5_ref[:], b5_ref[:],
                   None, None, None)
        for g in range(_G)
    ]
    for g in range(_G):
        xc = jnp.concatenate([x1[g], x2[g], x3[g]], axis=1)
        h6 = _lrelu(_dotd(xc, w6_ref[:]) * g6_ref[:] + b6_ref[:])
        g6 = jnp.max(h6, axis=0, keepdims=True)
        h7 = _lrelu(
            (_dotd(g6, w7g_ref[:]) + _dotd(xc, w7abc_ref[:]))
            * g7_ref[:] + b7_ref[:]
        )
        h8 = _lrelu(_dotd(h7, w8_ref[:]) * g8_ref[:] + b8_ref[:])
        out_ref[g] = _dotd(h8, w9_ref[:])


def kernel(obs, params):
    p = params
    s = 1.0 / math.sqrt(1.0 + _EPS_BN)

    def gs(name):
        return (p["g" + name] * s)[None, :]

    def bias(name):
        return p["b" + name][None, :]

    w0 = p["W0"].T
    w1 = p["W1"].T
    w2 = p["W2"].T
    w3 = p["W3"].T
    w4 = p["W4"].T
    w5 = p["W5"].T
    w6 = p["W6"].T
    w7 = p["W7"].T
    w8 = p["W8"].T
    w9 = p["W9"].T

    weights = [
        w0, gs("0"), bias("0"),
        w1[:32], w1[32:], gs("1"), bias("1"), w2, gs("2"), bias("2"),
        w3[:32], w3[32:], gs("3"), bias("3"), w4, gs("4"), bias("4"),
        w5[:32], w5[32:], gs("5"), bias("5"),
        w6, gs("6"), bias("6"),
        w7[:512], w7[512:], gs("7"), bias("7"),
        w8, gs("8"), bias("8"), w9,
    ]

    b = obs.shape[0]
    obs3 = obs.reshape(b, _N, obs.shape[1] // _N)

    in_specs = [pl.BlockSpec((_G, _N, obs3.shape[2]), lambda i: (i, 0, 0))]
    for w in weights:
        in_specs.append(
            pl.BlockSpec(w.shape, (lambda nd: (lambda i: (0,) * nd))(w.ndim))
        )

    out = pl.pallas_call(
        _dgcnn_kernel,
        grid=(b // _G,),
        in_specs=in_specs,
        out_specs=pl.BlockSpec((_G, _N, 1), lambda i: (i, 0, 0)),
        out_shape=jax.ShapeDtypeStruct((b, _N, 1), jnp.float32),
        compiler_params=pltpu.CompilerParams(
            dimension_semantics=("parallel",)
        ),
    )(obs3, *weights)

    q = out.reshape(b, _N)
    return q[None, :, None, :]
